# Initial kernel scaffold; baseline (speedup 1.0000x reference)
import jax
import jax.numpy as jnp
from jax.experimental import pallas as pl


def kernel(out_sample_hat, indices, empirical_distribution):
    raise NotImplementedError("write your pallas kernel here")


# speedup vs baseline: 3.3174x; 3.3174x over previous
import functools

import jax
import jax.numpy as jnp
from jax import lax
from jax.experimental import pallas as pl
from jax.experimental.pallas import tpu as pltpu


def _stage(arrs, j, k, sp, lt):
    g = sp // (2 * j)
    n = arrs[0].shape[-1]
    halves = [a.reshape(g, 2, j, n) for a in arrs]
    ah = [h[:, 0] for h in halves]
    bh = [h[:, 1] for h in halves]
    gi = lax.broadcasted_iota(jnp.int32, (g, 1, 1), 0)
    asc = ((gi * (2 * j)) & k) == 0
    ltab = lt(ah, bh)
    keep = ltab == asc
    outs = []
    for a, b in zip(ah, bh):
        na = jnp.where(keep, a, b)
        nb = jnp.where(keep, b, a)
        outs.append(
            jnp.concatenate([na[:, None], nb[:, None]], axis=1).reshape(sp, n)
        )
    return outs


def _bitonic(arrs, sp, lt):
    k = 2
    while k <= sp:
        j = k // 2
        while j >= 1:
            arrs = _stage(arrs, j, k, sp, lt)
            j //= 2
        k *= 2
    return arrs


def _body(idx_ref, x_ref, emp_ref, out_ref, ysel_ref, *, s, sp, n):
    b = pl.program_id(0)

    def gather_row(i, carry):
        t = idx_ref[b, i]
        ysel_ref[pl.ds(i, 1), :] = emp_ref[pl.ds(t, 1), :]
        return carry

    lax.fori_loop(0, s, gather_row, 0, unroll=8)

    inf = jnp.float32(jnp.inf)
    row = lax.broadcasted_iota(jnp.int32, (sp, n), 0)

    xt = x_ref[0].T
    xpad = jnp.concatenate([xt, jnp.full((sp - s, n), inf, jnp.float32)], axis=0)
    (sx,) = _bitonic([xpad], sp, lambda a, b: a[0] < b[0])

    y = jnp.where(row < s, ysel_ref[...], inf)
    _, p = _bitonic(
        [y, row],
        sp,
        lambda a, b: (a[0] < b[0]) | ((a[0] == b[0]) & (a[1] < b[1])),
    )

    _, out_t = _bitonic([p, sx], sp, lambda a, b: a[0] < b[0])

    out_ref[0] = out_t[:s].T


def kernel(out_sample_hat, indices, empirical_distribution):
    bsz, n, s = out_sample_hat.shape
    t = empirical_distribution.shape[0]
    sp = max(8, 1 << (s - 1).bit_length())

    body = functools.partial(_body, s=s, sp=sp, n=n)
    grid_spec = pltpu.PrefetchScalarGridSpec(
        num_scalar_prefetch=1,
        grid=(bsz,),
        in_specs=[
            pl.BlockSpec((1, n, s), lambda b, idx: (b, 0, 0)),
            pl.BlockSpec((t, n), lambda b, idx: (0, 0)),
        ],
        out_specs=pl.BlockSpec((1, n, s), lambda b, idx: (b, 0, 0)),
        scratch_shapes=[pltpu.VMEM((sp, n), jnp.float32)],
    )
    return pl.pallas_call(
        body,
        grid_spec=grid_spec,
        out_shape=jax.ShapeDtypeStruct((bsz, n, s), out_sample_hat.dtype),
        compiler_params=pltpu.CompilerParams(
            dimension_semantics=("arbitrary",),
            vmem_limit_bytes=110 * 1024 * 1024,
        ),
    )(indices.astype(jnp.int32), out_sample_hat, empirical_distribution)


# speedup vs baseline: 4.2137x; 1.2702x over previous
import functools

import jax
import jax.numpy as jnp
from jax import lax
from jax.experimental import pallas as pl
from jax.experimental.pallas import tpu as pltpu


def _swap(arrs, j, keep):
    sp = arrs[0].shape[0]
    n = arrs[0].shape[-1]
    g = sp // (2 * j)
    outs = []
    for arr in arrs:
        h = arr.reshape(g, 2, j, n)
        a, b = h[:, 0], h[:, 1]
        na = jnp.where(keep, a, b)
        nb = jnp.where(keep, b, a)
        outs.append(
            jnp.concatenate([na[:, None], nb[:, None]], axis=1).reshape(sp, n)
        )
    return outs


def _stage_masks(arrs, j, k, sp, lt):
    g = sp // (2 * j)
    n = arrs[0].shape[-1]
    halves = [a.reshape(g, 2, j, n) for a in arrs]
    ah = [h[:, 0] for h in halves]
    bh = [h[:, 1] for h in halves]
    gi = lax.broadcasted_iota(jnp.int32, (g, 1, 1), 0)
    asc = ((gi * (2 * j)) & k) == 0
    keep = lt(ah, bh) == asc
    return _swap(arrs, j, keep), keep


def _bitonic(arrs, sp, lt):
    stages = []
    k = 2
    while k <= sp:
        j = k // 2
        while j >= 1:
            arrs, keep = _stage_masks(arrs, j, k, sp, lt)
            stages.append((j, keep))
            j //= 2
        k *= 2
    return arrs, stages


def _unsort(arr, stages):
    for j, keep in reversed(stages):
        (arr,) = _swap([arr], j, keep)
    return arr


def _body(idx_ref, x_ref, emp_ref, out_ref, ysel_ref, *, s, sp, n):
    b = pl.program_id(0)

    def gather_row(i, carry):
        t = idx_ref[b, i]
        ysel_ref[pl.ds(i, 1), :] = emp_ref[pl.ds(t, 1), :]
        return carry

    lax.fori_loop(0, s, gather_row, 0, unroll=8)

    inf = jnp.float32(jnp.inf)
    row = lax.broadcasted_iota(jnp.int32, (sp, n), 0)

    xt = x_ref[0].T
    xpad = jnp.concatenate([xt, jnp.full((sp - s, n), inf, jnp.float32)], axis=0)
    (sx,), _ = _bitonic([xpad], sp, lambda a, b: a[0] < b[0])

    y = jnp.where(row < s, ysel_ref[...], inf)
    _, stages = _bitonic(
        [y, row],
        sp,
        lambda a, b: (a[0] < b[0]) | ((a[0] == b[0]) & (a[1] < b[1])),
    )

    out_t = _unsort(sx, stages)

    out_ref[0] = out_t[:s].T


def kernel(out_sample_hat, indices, empirical_distribution):
    bsz, n, s = out_sample_hat.shape
    t = empirical_distribution.shape[0]
    sp = max(8, 1 << (s - 1).bit_length())

    body = functools.partial(_body, s=s, sp=sp, n=n)
    grid_spec = pltpu.PrefetchScalarGridSpec(
        num_scalar_prefetch=1,
        grid=(bsz,),
        in_specs=[
            pl.BlockSpec((1, n, s), lambda b, idx: (b, 0, 0)),
            pl.BlockSpec((t, n), lambda b, idx: (0, 0)),
        ],
        out_specs=pl.BlockSpec((1, n, s), lambda b, idx: (b, 0, 0)),
        scratch_shapes=[pltpu.VMEM((sp, n), jnp.float32)],
    )
    return pl.pallas_call(
        body,
        grid_spec=grid_spec,
        out_shape=jax.ShapeDtypeStruct((bsz, n, s), out_sample_hat.dtype),
        compiler_params=pltpu.CompilerParams(
            dimension_semantics=("arbitrary",),
            vmem_limit_bytes=110 * 1024 * 1024,
        ),
    )(indices.astype(jnp.int32), out_sample_hat, empirical_distribution)
